# Initial kernel scaffold; baseline (speedup 1.0000x reference)
import functools

import jax
import jax.numpy as jnp
from jax import lax
from jax.experimental import pallas as pl
from jax.experimental.pallas import tpu as pltpu
from jax.experimental.pallas import tpu_sc as plsc

N = 10000
E = 320000
P = 100000
IN_CH = 128
HIDDEN = 256

NC = 2
NS = 16
NPAD = 10240
RPT = NPAD // NS
F = 64
K_SEG = 400
EPT = E // NS
PPAD = 102400
PPT = PPAD // (NC * NS)
K_DEC = 160


def _fill(ref, val, rows, cols):
    v = jnp.full((16,), val, jnp.float32)

    def row(r, carry):
        def col(k, carry2):
            ref[r, pl.ds(k * 16, 16)] = v
            return carry2
        return lax.fori_loop(0, cols // 16, col, carry)

    lax.fori_loop(0, rows, row, 0)


def _make_segsum(nq, with_cnt):
    qpc = nq // NC
    mesh = plsc.VectorSubcoreMesh(core_axis_name="c", subcore_axis_name="s")

    out_type = [jax.ShapeDtypeStruct((nq, NPAD, F), jnp.float32)]
    if with_cnt:
        out_type.append(jax.ShapeDtypeStruct((NPAD, 16), jnp.float32))

    scratch = dict(
        tab_s=pltpu.VMEM_SHARED((NPAD, F), jnp.float32),
        acc_s=pltpu.VMEM_SHARED((NPAD, F), jnp.float32),
        sidx=pltpu.VMEM((K_SEG,), jnp.int32),
        didx=pltpu.VMEM((K_SEG,), jnp.int32),
        rows_v=pltpu.VMEM((K_SEG, F), jnp.float32),
        zer=pltpu.VMEM((64, F), jnp.float32),
        sem=pltpu.SemaphoreType.DMA,
    )
    if with_cnt:
        scratch.update(
            cnt_s=pltpu.VMEM_SHARED((NPAD, 16), jnp.float32),
            ones_v=pltpu.VMEM((K_SEG, 16), jnp.float32),
            zer16=pltpu.VMEM((64, 16), jnp.float32),
        )

    def body(tab_hbm, src_hbm, dst_hbm, out_hbm, *rest):
        if with_cnt:
            (cnt_hbm, tab_s, acc_s, sidx, didx, rows_v, zer, sem,
             cnt_s, ones_v, zer16) = rest
        else:
            (tab_s, acc_s, sidx, didx, rows_v, zer, sem) = rest
        c = lax.axis_index("c")
        s = lax.axis_index("s")
        r0 = s * RPT

        _fill(zer, 0.0, 64, F)
        if with_cnt:
            _fill(zer16, 0.0, 64, 16)
            _fill(ones_v, 1.0, K_SEG, 16)

        for qi in range(qpc):
            q = c * qpc + qi
            pltpu.sync_copy(tab_hbm.at[q, pl.ds(r0, RPT)],
                            tab_s.at[pl.ds(r0, RPT)])
            for zb in range(RPT // 64):
                pltpu.sync_copy(zer, acc_s.at[pl.ds(r0 + zb * 64, 64)])
            if with_cnt and qi == 0:
                @pl.when(c == 0)
                def _():
                    for zb in range(RPT // 64):
                        pltpu.sync_copy(zer16,
                                        cnt_s.at[pl.ds(r0 + zb * 64, 64)])
            plsc.subcore_barrier()

            def chunk(j, carry):
                base = s * EPT + j * K_SEG
                pltpu.sync_copy(src_hbm.at[pl.ds(base, K_SEG)], sidx)
                pltpu.async_copy(tab_s.at[sidx], rows_v, sem).wait()
                pltpu.sync_copy(dst_hbm.at[pl.ds(base, K_SEG)], didx)
                pltpu.sync_copy(rows_v, acc_s.at[didx], add=True)
                if with_cnt and qi == 0:
                    @pl.when(c == 0)
                    def _():
                        pltpu.sync_copy(ones_v, cnt_s.at[didx], add=True)
                return carry

            lax.fori_loop(0, EPT // K_SEG, chunk, 0)
            plsc.subcore_barrier()

            pltpu.sync_copy(acc_s.at[pl.ds(r0, RPT)],
                            out_hbm.at[q, pl.ds(r0, RPT)])
            if with_cnt and qi == 0:
                @pl.when(c == 0)
                def _():
                    pltpu.sync_copy(cnt_s.at[pl.ds(r0, RPT)],
                                    cnt_hbm.at[pl.ds(r0, RPT)])

    return pl.kernel(body, out_type=tuple(out_type), mesh=mesh,
                     scratch_types=scratch)


_segsum2 = _make_segsum(2, True)
_segsum4 = _make_segsum(4, False)


RB = 512
NB = NPAD // RB


def _combine1_body(agg_ref, cnt_ref, x_ref, wl_ref, b_ref, wr_ref, out_ref):
    cnt = jnp.maximum(cnt_ref[:, 0:1], 1.0)
    mean = jnp.concatenate([agg_ref[0], agg_ref[1]], axis=-1) / cnt
    h = (jnp.dot(mean, wl_ref[...], preferred_element_type=jnp.float32)
         + b_ref[...]
         + jnp.dot(x_ref[...], wr_ref[...],
                   preferred_element_type=jnp.float32))
    h = jnp.maximum(h, 0.0)
    for q in range(4):
        out_ref[q] = h[:, q * F:(q + 1) * F]


def _combine1(agg1, cnt16, x_pad, W1_l, b1, W1_r):
    return pl.pallas_call(
        _combine1_body,
        grid=(NB,),
        in_specs=[
            pl.BlockSpec((2, RB, F), lambda i: (0, i, 0)),
            pl.BlockSpec((RB, 16), lambda i: (i, 0)),
            pl.BlockSpec((RB, IN_CH), lambda i: (i, 0)),
            pl.BlockSpec((IN_CH, HIDDEN), lambda i: (0, 0)),
            pl.BlockSpec((1, HIDDEN), lambda i: (0, 0)),
            pl.BlockSpec((IN_CH, HIDDEN), lambda i: (0, 0)),
        ],
        out_specs=pl.BlockSpec((4, RB, F), lambda i: (0, i, 0)),
        out_shape=jax.ShapeDtypeStruct((4, NPAD, F), jnp.float32),
    )(agg1, cnt16, x_pad, W1_l, b1.reshape(1, HIDDEN), W1_r)


def _combine2_body(agg_ref, cnt_ref, h_ref, wl_ref, b_ref, wr_ref, out_ref):
    cnt = jnp.maximum(cnt_ref[:, 0:1], 1.0)
    mean = jnp.concatenate([agg_ref[q] for q in range(4)], axis=-1) / cnt
    h = jnp.concatenate([h_ref[q] for q in range(4)], axis=-1)
    out_ref[...] = (
        jnp.dot(mean, wl_ref[...], preferred_element_type=jnp.float32)
        + b_ref[...]
        + jnp.dot(h, wr_ref[...], preferred_element_type=jnp.float32))


def _combine2(agg2, cnt16, hT2, W2_l, b2, W2_r):
    return pl.pallas_call(
        _combine2_body,
        grid=(NB,),
        in_specs=[
            pl.BlockSpec((4, RB, F), lambda i: (0, i, 0)),
            pl.BlockSpec((RB, 16), lambda i: (i, 0)),
            pl.BlockSpec((4, RB, F), lambda i: (0, i, 0)),
            pl.BlockSpec((HIDDEN, HIDDEN), lambda i: (0, 0)),
            pl.BlockSpec((1, HIDDEN), lambda i: (0, 0)),
            pl.BlockSpec((HIDDEN, HIDDEN), lambda i: (0, 0)),
        ],
        out_specs=pl.BlockSpec((RB, HIDDEN), lambda i: (i, 0)),
        out_shape=jax.ShapeDtypeStruct((NPAD, HIDDEN), jnp.float32),
    )(agg2, cnt16, hT2, W2_l, b2.reshape(1, HIDDEN), W2_r)


def _make_decode():
    mesh = plsc.VectorSubcoreMesh(core_axis_name="c", subcore_axis_name="s")
    scratch = dict(
        sidx=pltpu.VMEM((K_DEC,), jnp.int32),
        didx=pltpu.VMEM((K_DEC,), jnp.int32),
        zs=pltpu.VMEM((K_DEC, HIDDEN), jnp.float32),
        zd=pltpu.VMEM((K_DEC, HIDDEN), jnp.float32),
        outv=pltpu.VMEM((K_DEC,), jnp.float32),
        sem1=pltpu.SemaphoreType.DMA,
        sem2=pltpu.SemaphoreType.DMA,
    )

    def body(z_hbm, es_hbm, ed_hbm, out_hbm, sidx, didx, zs, zd, outv,
             sem1, sem2):
        c = lax.axis_index("c")
        s = lax.axis_index("s")
        w = c * NS + s
        lanes = lax.iota(jnp.int32, 16)

        def chunk(j, carry):
            base = w * PPT + j * K_DEC
            pltpu.sync_copy(es_hbm.at[pl.ds(base, K_DEC)], sidx)
            cp1 = pltpu.async_copy(z_hbm.at[sidx], zs, sem1)
            pltpu.sync_copy(ed_hbm.at[pl.ds(base, K_DEC)], didx)
            cp2 = pltpu.async_copy(z_hbm.at[didx], zd, sem2)
            cp1.wait()
            cp2.wait()
            for g in range(K_DEC // 16):
                rows16 = lanes + (g * 16)

                def feat(k, acc):
                    kk = jnp.full((16,), 0, jnp.int32) + k
                    a = plsc.load_gather(zs, [rows16, kk])
                    b = plsc.load_gather(zd, [rows16, kk])
                    return acc + a * b

                acc = lax.fori_loop(0, HIDDEN, feat,
                                    jnp.zeros((16,), jnp.float32))
                outv[pl.ds(g * 16, 16)] = acc
            pltpu.sync_copy(outv, out_hbm.at[pl.ds(base, K_DEC)])
            return carry

        lax.fori_loop(0, PPT // K_DEC, chunk, 0)

    return pl.kernel(body,
                     out_type=jax.ShapeDtypeStruct((PPAD,), jnp.float32),
                     mesh=mesh, scratch_types=scratch)


_decode = _make_decode()


def kernel(x, edge_index, edges, W1_l, b1, W1_r, W2_l, b2, W2_r):
    src = edge_index[0]
    dst = edge_index[1]
    x_pad = jnp.pad(x, ((0, NPAD - N), (0, 0)))
    xT2 = x_pad.reshape(NPAD, 2, F).transpose(1, 0, 2)

    agg1, cnt16 = _segsum2(xT2, src, dst)
    hT2 = _combine1(agg1, cnt16, x_pad, W1_l, b1, W1_r)
    agg2 = _segsum4(hT2, src, dst)
    z = _combine2(agg2, cnt16, hT2, W2_l, b2, W2_r)

    es = jnp.pad(edges[:, 0], (0, PPAD - P))
    ed = jnp.pad(edges[:, 1], (0, PPAD - P))
    out = _decode(z, es, ed)
    return out[:P]


# speedup vs baseline: 2.1287x; 2.1287x over previous
import functools

import jax
import jax.numpy as jnp
from jax import lax
from jax.experimental import pallas as pl
from jax.experimental.pallas import tpu as pltpu
from jax.experimental.pallas import tpu_sc as plsc

N = 10000
E = 320000
P = 100000
IN_CH = 128
HIDDEN = 256

NC = 2
NS = 16
NPAD = 10240
RPT = NPAD // NS
F = 64
K_SEG = 80
EPT = E // NS
PPAD = 102400
PPT = PPAD // (NC * NS)
K_DEC = 128


def _fill(ref, val, rows, cols):
    v = jnp.full((16,), val, jnp.float32)

    def row(r, carry):
        def col(k, carry2):
            ref[r, pl.ds(k * 16, 16)] = v
            return carry2
        return lax.fori_loop(0, cols // 16, col, carry)

    lax.fori_loop(0, rows, row, 0)


def _fill_1d(ref, val, n):
    v = jnp.full((16,), val, jnp.float32)

    def it(k, carry):
        ref[pl.ds(k * 16, 16)] = v
        return carry

    lax.fori_loop(0, n // 16, it, 0)


def _make_segsum(nq, with_cnt):
    qpc = nq // NC
    mesh = plsc.VectorSubcoreMesh(core_axis_name="c", subcore_axis_name="s")

    out_type = [jax.ShapeDtypeStruct((nq, NPAD, F), jnp.float32)]
    if with_cnt:
        out_type.append(jax.ShapeDtypeStruct((NPAD,), jnp.float32))

    scratch = [
        pltpu.VMEM_SHARED((NPAD, F), jnp.float32),
        pltpu.VMEM_SHARED((NPAD, F), jnp.float32),
        pltpu.VMEM((K_SEG,), jnp.int32),
        pltpu.VMEM((K_SEG,), jnp.int32),
        pltpu.VMEM((K_SEG, F), jnp.float32),
        pltpu.SemaphoreType.DMA,
    ]
    if with_cnt:
        scratch += [
            pltpu.VMEM_SHARED((NPAD,), jnp.float32),
            pltpu.VMEM((K_SEG,), jnp.float32),
        ]

    def body(tab_hbm, src_hbm, dst_hbm, out_hbm, *rest):
        if with_cnt:
            (cnt_hbm, tab_s, acc_s, sidx, didx, rows_v, sem,
             cnt_s, ones_v) = rest
        else:
            (tab_s, acc_s, sidx, didx, rows_v, sem) = rest
        c = lax.axis_index("c")
        s = lax.axis_index("s")
        r0 = s * RPT

        nzc = RPT // K_SEG
        rem = RPT - nzc * K_SEG

        for qi in range(qpc):
            q = c * qpc + qi
            _fill(rows_v, 0.0, K_SEG, F)
            pltpu.sync_copy(tab_hbm.at[q, pl.ds(r0, RPT)],
                            tab_s.at[pl.ds(r0, RPT)])
            for zb in range(nzc):
                pltpu.sync_copy(rows_v,
                                acc_s.at[pl.ds(r0 + zb * K_SEG, K_SEG)])
            if rem:
                pltpu.sync_copy(rows_v.at[pl.ds(0, rem)],
                                acc_s.at[pl.ds(r0 + nzc * K_SEG, rem)])
            if with_cnt and qi == 0:
                _fill_1d(ones_v, 0.0, K_SEG)

                @pl.when(c == 0)
                def _():
                    for zb in range(nzc):
                        pltpu.sync_copy(
                            ones_v, cnt_s.at[pl.ds(r0 + zb * K_SEG, K_SEG)])
                    if rem:
                        pltpu.sync_copy(
                            ones_v.at[pl.ds(0, rem)],
                            cnt_s.at[pl.ds(r0 + nzc * K_SEG, rem)])
                _fill_1d(ones_v, 1.0, K_SEG)
            plsc.subcore_barrier()

            def chunk(j, carry):
                base = s * EPT + j * K_SEG
                pltpu.sync_copy(src_hbm.at[pl.ds(base, K_SEG)], sidx)
                pltpu.async_copy(tab_s.at[sidx], rows_v, sem).wait()
                pltpu.sync_copy(dst_hbm.at[pl.ds(base, K_SEG)], didx)
                pltpu.sync_copy(rows_v, acc_s.at[didx], add=True)
                if with_cnt and qi == 0:
                    @pl.when(c == 0)
                    def _():
                        pltpu.sync_copy(ones_v, cnt_s.at[didx], add=True)
                return carry

            lax.fori_loop(0, EPT // K_SEG, chunk, 0)
            plsc.subcore_barrier()

            pltpu.sync_copy(acc_s.at[pl.ds(r0, RPT)],
                            out_hbm.at[q, pl.ds(r0, RPT)])
            if with_cnt and qi == 0:
                @pl.when(c == 0)
                def _():
                    pltpu.sync_copy(cnt_s.at[pl.ds(r0, RPT)],
                                    cnt_hbm.at[pl.ds(r0, RPT)])

    return pl.kernel(body, out_type=tuple(out_type), mesh=mesh,
                     scratch_types=scratch,
                     compiler_params=pltpu.CompilerParams(
                         use_tc_tiling_on_sc=False))


_segsum2 = _make_segsum(2, True)
_segsum4 = _make_segsum(4, False)


RB = 512
NB = NPAD // RB


def _combine1_body(agg_ref, cnt_ref, x_ref, wl_ref, b_ref, wr_ref, out_ref):
    cnt = jnp.maximum(cnt_ref[...], 1.0)
    mean = jnp.concatenate([agg_ref[0], agg_ref[1]], axis=-1) / cnt
    h = (jnp.dot(mean, wl_ref[...], preferred_element_type=jnp.float32)
         + b_ref[...]
         + jnp.dot(x_ref[...], wr_ref[...],
                   preferred_element_type=jnp.float32))
    h = jnp.maximum(h, 0.0)
    for q in range(4):
        out_ref[q] = h[:, q * F:(q + 1) * F]


def _combine1(agg1, cnt16, x_pad, W1_l, b1, W1_r):
    return pl.pallas_call(
        _combine1_body,
        grid=(NB,),
        in_specs=[
            pl.BlockSpec((2, RB, F), lambda i: (0, i, 0)),
            pl.BlockSpec((RB, 1), lambda i: (i, 0)),
            pl.BlockSpec((RB, IN_CH), lambda i: (i, 0)),
            pl.BlockSpec((IN_CH, HIDDEN), lambda i: (0, 0)),
            pl.BlockSpec((1, HIDDEN), lambda i: (0, 0)),
            pl.BlockSpec((IN_CH, HIDDEN), lambda i: (0, 0)),
        ],
        out_specs=pl.BlockSpec((4, RB, F), lambda i: (0, i, 0)),
        out_shape=jax.ShapeDtypeStruct((4, NPAD, F), jnp.float32),
    )(agg1, cnt16, x_pad, W1_l, b1.reshape(1, HIDDEN), W1_r)


def _combine2_body(agg_ref, cnt_ref, h_ref, wl_ref, b_ref, wr_ref, out_ref):
    cnt = jnp.maximum(cnt_ref[...], 1.0)
    mean = jnp.concatenate([agg_ref[q] for q in range(4)], axis=-1) / cnt
    h = jnp.concatenate([h_ref[q] for q in range(4)], axis=-1)
    out_ref[...] = (
        jnp.dot(mean, wl_ref[...], preferred_element_type=jnp.float32)
        + b_ref[...]
        + jnp.dot(h, wr_ref[...], preferred_element_type=jnp.float32))


def _combine2(agg2, cnt16, hT2, W2_l, b2, W2_r):
    return pl.pallas_call(
        _combine2_body,
        grid=(NB,),
        in_specs=[
            pl.BlockSpec((4, RB, F), lambda i: (0, i, 0)),
            pl.BlockSpec((RB, 1), lambda i: (i, 0)),
            pl.BlockSpec((4, RB, F), lambda i: (0, i, 0)),
            pl.BlockSpec((HIDDEN, HIDDEN), lambda i: (0, 0)),
            pl.BlockSpec((1, HIDDEN), lambda i: (0, 0)),
            pl.BlockSpec((HIDDEN, HIDDEN), lambda i: (0, 0)),
        ],
        out_specs=pl.BlockSpec((RB, HIDDEN), lambda i: (i, 0)),
        out_shape=jax.ShapeDtypeStruct((NPAD, HIDDEN), jnp.float32),
    )(agg2, cnt16, hT2, W2_l, b2.reshape(1, HIDDEN), W2_r)


def _make_decode():
    mesh = plsc.VectorSubcoreMesh(core_axis_name="c", subcore_axis_name="s")
    scratch = [
        pltpu.VMEM((K_DEC,), jnp.int32),
        pltpu.VMEM((K_DEC,), jnp.int32),
        pltpu.VMEM((K_DEC, HIDDEN), jnp.float32),
        pltpu.VMEM((K_DEC, HIDDEN), jnp.float32),
        pltpu.VMEM((K_DEC,), jnp.float32),
        pltpu.SemaphoreType.DMA,
        pltpu.SemaphoreType.DMA,
    ]

    def body(z_hbm, es_hbm, ed_hbm, out_hbm, sidx, didx, zs, zd, outv,
             sem1, sem2):
        c = lax.axis_index("c")
        s = lax.axis_index("s")
        w = c * NS + s
        lanes = lax.iota(jnp.int32, 16)

        def chunk(j, carry):
            base = w * PPT + j * K_DEC
            pltpu.sync_copy(es_hbm.at[pl.ds(base, K_DEC)], sidx)
            cp1 = pltpu.async_copy(z_hbm.at[sidx], zs, sem1)
            pltpu.sync_copy(ed_hbm.at[pl.ds(base, K_DEC)], didx)
            cp2 = pltpu.async_copy(z_hbm.at[didx], zd, sem2)
            cp1.wait()
            cp2.wait()
            for g in range(K_DEC // 16):
                rows16 = lanes + (g * 16)

                def feat(k, acc):
                    kk = jnp.full((16,), 0, jnp.int32) + k
                    a = plsc.load_gather(zs, [rows16, kk])
                    b = plsc.load_gather(zd, [rows16, kk])
                    return acc + a * b

                acc = lax.fori_loop(0, HIDDEN, feat,
                                    jnp.zeros((16,), jnp.float32))
                outv[pl.ds(g * 16, 16)] = acc
            pltpu.sync_copy(outv, out_hbm.at[pl.ds(base, K_DEC)])
            return carry

        lax.fori_loop(0, PPT // K_DEC, chunk, 0)

    return pl.kernel(body,
                     out_type=jax.ShapeDtypeStruct((PPAD,), jnp.float32),
                     mesh=mesh, scratch_types=scratch,
                     compiler_params=pltpu.CompilerParams(
                         use_tc_tiling_on_sc=False,
                         needs_layout_passes=False))


_decode = _make_decode()


def kernel(x, edge_index, edges, W1_l, b1, W1_r, W2_l, b2, W2_r):
    src = edge_index[0]
    dst = edge_index[1]
    x_pad = jnp.pad(x, ((0, NPAD - N), (0, 0)))
    xT2 = x_pad.reshape(NPAD, 2, F).transpose(1, 0, 2)

    agg1, cnt = _segsum2(xT2, src, dst)
    cnt2d = cnt.reshape(NPAD, 1)
    hT2 = _combine1(agg1, cnt2d, x_pad, W1_l, b1, W1_r)
    (agg2,) = _segsum4(hT2, src, dst)
    z = _combine2(agg2, cnt2d, hT2, W2_l, b2, W2_r)

    es = jnp.pad(edges[:, 0], (0, PPAD - P))
    ed = jnp.pad(edges[:, 1], (0, PPAD - P))
    out = _decode(z, es, ed)
    return out[:P]


# speedup vs baseline: 2.5426x; 1.1945x over previous
import functools

import jax
import jax.numpy as jnp
from jax import lax
from jax.experimental import pallas as pl
from jax.experimental.pallas import tpu as pltpu
from jax.experimental.pallas import tpu_sc as plsc

N = 10000
E = 320000
P = 100000
IN_CH = 128
HIDDEN = 256

NC = 2
NS = 16
NPAD = 10240
RPT = NPAD // NS
F = 64

K_SEG = 80
EPT = E // NS
CPT = EPT // K_SEG
SJ = 50
SB = CPT // SJ

K_DEC = 96
DCH = 34
PPT = K_DEC * DCH
PPAD = PPT * NC * NS


def _fill(ref, val, rows, cols):
    v = jnp.full((16,), val, jnp.float32)

    def row(r, carry):
        def col(k, carry2):
            ref[r, pl.ds(k * 16, 16)] = v
            return carry2
        return lax.fori_loop(0, cols // 16, col, carry)

    lax.fori_loop(0, rows, row, 0)


def _fill_1d(ref, val, n):
    v = jnp.full((16,), val, jnp.float32)

    def it(k, carry):
        ref[pl.ds(k * 16, 16)] = v
        return carry

    lax.fori_loop(0, n // 16, it, 0)


def _make_segsum(nq, with_cnt):
    qpc = nq // NC
    mesh = plsc.VectorSubcoreMesh(core_axis_name="c", subcore_axis_name="s")

    out_type = [jax.ShapeDtypeStruct((nq, NPAD, F), jnp.float32)]
    if with_cnt:
        out_type.append(jax.ShapeDtypeStruct((NPAD,), jnp.float32))

    scratch = [
        pltpu.VMEM_SHARED((NPAD, F), jnp.float32),
        pltpu.VMEM_SHARED((NPAD, F), jnp.float32),
        pltpu.VMEM((SJ, K_SEG), jnp.int32),
        pltpu.VMEM((SJ, K_SEG), jnp.int32),
        pltpu.VMEM((K_SEG, F), jnp.float32),
        pltpu.VMEM((K_SEG, F), jnp.float32),
        pltpu.SemaphoreType.DMA,
        pltpu.SemaphoreType.DMA,
        pltpu.SemaphoreType.DMA,
        pltpu.SemaphoreType.DMA,
    ]
    if with_cnt:
        scratch += [
            pltpu.VMEM_SHARED((NPAD,), jnp.float32),
            pltpu.VMEM((K_SEG,), jnp.float32),
            pltpu.SemaphoreType.DMA,
            pltpu.SemaphoreType.DMA,
        ]

    def body(tab_hbm, src_hbm, dst_hbm, out_hbm, *rest):
        if with_cnt:
            (cnt_hbm, tab_s, acc_s, sidx_blk, didx_blk, rows_v0, rows_v1,
             semg0, semg1, sems0, sems1, cnt_s, ones_v, semc0, semc1) = rest
        else:
            (tab_s, acc_s, sidx_blk, didx_blk, rows_v0, rows_v1,
             semg0, semg1, sems0, sems1) = rest
        c = lax.axis_index("c")
        s = lax.axis_index("s")
        r0 = s * RPT

        for qi in range(qpc):
            q = c * qpc + qi
            _fill(rows_v0, 0.0, K_SEG, F)
            pltpu.sync_copy(tab_hbm.at[q, pl.ds(r0, RPT)],
                            tab_s.at[pl.ds(r0, RPT)])
            for zb in range(RPT // K_SEG):
                pltpu.sync_copy(rows_v0,
                                acc_s.at[pl.ds(r0 + zb * K_SEG, K_SEG)])
            if with_cnt and qi == 0:
                _fill_1d(ones_v, 0.0, K_SEG)

                @pl.when(c == 0)
                def _():
                    for zb in range(RPT // K_SEG):
                        pltpu.sync_copy(
                            ones_v, cnt_s.at[pl.ds(r0 + zb * K_SEG, K_SEG)])
                _fill_1d(ones_v, 1.0, K_SEG)
            plsc.subcore_barrier()

            def sblk(u, carry):
                row0 = s * CPT + u * SJ
                pltpu.sync_copy(src_hbm.at[pl.ds(row0, SJ)], sidx_blk)
                pltpu.sync_copy(dst_hbm.at[pl.ds(row0, SJ)], didx_blk)

                def pair(i, carry2):
                    cpg0 = pltpu.async_copy(
                        tab_s.at[sidx_blk.at[2 * i]], rows_v0, semg0)
                    cpg1 = pltpu.async_copy(
                        tab_s.at[sidx_blk.at[2 * i + 1]], rows_v1, semg1)
                    cpg0.wait()
                    cps0 = pltpu.async_copy(
                        rows_v0, acc_s.at[didx_blk.at[2 * i]], sems0,
                        add=True)
                    cpg1.wait()
                    cps1 = pltpu.async_copy(
                        rows_v1, acc_s.at[didx_blk.at[2 * i + 1]], sems1,
                        add=True)
                    if with_cnt and qi == 0:
                        @pl.when(c == 0)
                        def _():
                            cpc0 = pltpu.async_copy(
                                ones_v, cnt_s.at[didx_blk.at[2 * i]],
                                semc0, add=True)
                            cpc1 = pltpu.async_copy(
                                ones_v, cnt_s.at[didx_blk.at[2 * i + 1]],
                                semc1, add=True)
                            cpc0.wait()
                            cpc1.wait()
                    cps0.wait()
                    cps1.wait()
                    return carry2

                lax.fori_loop(0, SJ // 2, pair, 0)
                return carry

            lax.fori_loop(0, SB, sblk, 0)
            plsc.subcore_barrier()

            pltpu.sync_copy(acc_s.at[pl.ds(r0, RPT)],
                            out_hbm.at[q, pl.ds(r0, RPT)])
            if with_cnt and qi == 0:
                @pl.when(c == 0)
                def _():
                    pltpu.sync_copy(cnt_s.at[pl.ds(r0, RPT)],
                                    cnt_hbm.at[pl.ds(r0, RPT)])

    return pl.kernel(body, out_type=tuple(out_type), mesh=mesh,
                     scratch_types=scratch,
                     compiler_params=pltpu.CompilerParams(
                         use_tc_tiling_on_sc=False))


_segsum2 = _make_segsum(2, True)
_segsum4 = _make_segsum(4, False)


RB = 512
NB = NPAD // RB


def _combine1_body(agg_ref, cnt_ref, x_ref, wl_ref, b_ref, wr_ref, out_ref):
    cnt = jnp.maximum(cnt_ref[...], 1.0)
    mean = jnp.concatenate([agg_ref[0], agg_ref[1]], axis=-1) / cnt
    h = (jnp.dot(mean, wl_ref[...], preferred_element_type=jnp.float32)
         + b_ref[...]
         + jnp.dot(x_ref[...], wr_ref[...],
                   preferred_element_type=jnp.float32))
    h = jnp.maximum(h, 0.0)
    for q in range(4):
        out_ref[q] = h[:, q * F:(q + 1) * F]


def _combine1(agg1, cnt2d, x_pad, W1_l, b1, W1_r):
    return pl.pallas_call(
        _combine1_body,
        grid=(NB,),
        in_specs=[
            pl.BlockSpec((2, RB, F), lambda i: (0, i, 0)),
            pl.BlockSpec((RB, 1), lambda i: (i, 0)),
            pl.BlockSpec((RB, IN_CH), lambda i: (i, 0)),
            pl.BlockSpec((IN_CH, HIDDEN), lambda i: (0, 0)),
            pl.BlockSpec((1, HIDDEN), lambda i: (0, 0)),
            pl.BlockSpec((IN_CH, HIDDEN), lambda i: (0, 0)),
        ],
        out_specs=pl.BlockSpec((4, RB, F), lambda i: (0, i, 0)),
        out_shape=jax.ShapeDtypeStruct((4, NPAD, F), jnp.float32),
    )(agg1, cnt2d, x_pad, W1_l, b1.reshape(1, HIDDEN), W1_r)


def _combine2_body(agg_ref, cnt_ref, h_ref, wl_ref, b_ref, wr_ref, out_ref):
    cnt = jnp.maximum(cnt_ref[...], 1.0)
    mean = jnp.concatenate([agg_ref[q] for q in range(4)], axis=-1) / cnt
    h = jnp.concatenate([h_ref[q] for q in range(4)], axis=-1)
    out_ref[...] = (
        jnp.dot(mean, wl_ref[...], preferred_element_type=jnp.float32)
        + b_ref[...]
        + jnp.dot(h, wr_ref[...], preferred_element_type=jnp.float32))


def _combine2(agg2, cnt2d, hT2, W2_l, b2, W2_r):
    return pl.pallas_call(
        _combine2_body,
        grid=(NB,),
        in_specs=[
            pl.BlockSpec((4, RB, F), lambda i: (0, i, 0)),
            pl.BlockSpec((RB, 1), lambda i: (i, 0)),
            pl.BlockSpec((4, RB, F), lambda i: (0, i, 0)),
            pl.BlockSpec((HIDDEN, HIDDEN), lambda i: (0, 0)),
            pl.BlockSpec((1, HIDDEN), lambda i: (0, 0)),
            pl.BlockSpec((HIDDEN, HIDDEN), lambda i: (0, 0)),
        ],
        out_specs=pl.BlockSpec((RB, HIDDEN), lambda i: (i, 0)),
        out_shape=jax.ShapeDtypeStruct((NPAD, HIDDEN), jnp.float32),
    )(agg2, cnt2d, hT2, W2_l, b2.reshape(1, HIDDEN), W2_r)


def _make_decode():
    mesh = plsc.VectorSubcoreMesh(core_axis_name="c", subcore_axis_name="s")
    scratch = [
        pltpu.VMEM((DCH, K_DEC), jnp.int32),
        pltpu.VMEM((DCH, K_DEC), jnp.int32),
        pltpu.VMEM((K_DEC, HIDDEN), jnp.float32),
        pltpu.VMEM((K_DEC, HIDDEN), jnp.float32),
        pltpu.VMEM((K_DEC, HIDDEN), jnp.float32),
        pltpu.VMEM((K_DEC, HIDDEN), jnp.float32),
        pltpu.VMEM((PPT,), jnp.float32),
        pltpu.SemaphoreType.DMA,
        pltpu.SemaphoreType.DMA,
        pltpu.SemaphoreType.DMA,
        pltpu.SemaphoreType.DMA,
    ]

    def body(z_hbm, es_hbm, ed_hbm, out_hbm, sidx_all, didx_all,
             zs0, zd0, zs1, zd1, outv, sem0a, sem0b, sem1a, sem1b):
        c = lax.axis_index("c")
        s = lax.axis_index("s")
        w = c * NS + s
        lanes = lax.iota(jnp.int32, 16)
        z16 = jnp.zeros((16,), jnp.float32)

        pltpu.sync_copy(es_hbm.at[pl.ds(w * DCH, DCH)], sidx_all)
        pltpu.sync_copy(ed_hbm.at[pl.ds(w * DCH, DCH)], didx_all)

        def compute(zs, zd, j):
            for g in range(K_DEC // 16):
                rows16 = lanes + (g * 16)

                def ki_loop(ki, accs):
                    a0, a1 = accs
                    kb = jnp.zeros((16,), jnp.int32) + ki * 16
                    for u in range(16):
                        kk = kb + u
                        va = plsc.load_gather(zs, [rows16, kk])
                        vb = plsc.load_gather(zd, [rows16, kk])
                        if u % 2 == 0:
                            a0 = a0 + va * vb
                        else:
                            a1 = a1 + va * vb
                    return (a0, a1)

                a0, a1 = lax.fori_loop(0, HIDDEN // 16, ki_loop, (z16, z16))
                outv[pl.ds(j * K_DEC + g * 16, 16)] = a0 + a1

        def it(i, carry):
            j0 = 2 * i
            j1 = 2 * i + 1
            cp0a = pltpu.async_copy(z_hbm.at[sidx_all.at[j0]], zs0, sem0a)
            cp0b = pltpu.async_copy(z_hbm.at[didx_all.at[j0]], zd0, sem0b)
            cp1a = pltpu.async_copy(z_hbm.at[sidx_all.at[j1]], zs1, sem1a)
            cp1b = pltpu.async_copy(z_hbm.at[didx_all.at[j1]], zd1, sem1b)
            cp0a.wait()
            cp0b.wait()
            compute(zs0, zd0, j0)
            cp1a.wait()
            cp1b.wait()
            compute(zs1, zd1, j1)
            return carry

        lax.fori_loop(0, DCH // 2, it, 0)

        pltpu.sync_copy(outv, out_hbm.at[pl.ds(w * PPT, PPT)])

    return pl.kernel(body,
                     out_type=jax.ShapeDtypeStruct((PPAD,), jnp.float32),
                     mesh=mesh, scratch_types=scratch,
                     compiler_params=pltpu.CompilerParams(
                         use_tc_tiling_on_sc=False,
                         needs_layout_passes=False))


_decode = _make_decode()


def kernel(x, edge_index, edges, W1_l, b1, W1_r, W2_l, b2, W2_r):
    src2 = edge_index[0].reshape(E // K_SEG, K_SEG)
    dst2 = edge_index[1].reshape(E // K_SEG, K_SEG)
    x_pad = jnp.pad(x, ((0, NPAD - N), (0, 0)))
    xT2 = x_pad.reshape(NPAD, 2, F).transpose(1, 0, 2)

    agg1, cnt = _segsum2(xT2, src2, dst2)
    cnt2d = cnt.reshape(NPAD, 1)
    hT2 = _combine1(agg1, cnt2d, x_pad, W1_l, b1, W1_r)
    (agg2,) = _segsum4(hT2, src2, dst2)
    z = _combine2(agg2, cnt2d, hT2, W2_l, b2, W2_r)

    es2 = jnp.pad(edges[:, 0], (0, PPAD - P)).reshape(PPAD // K_DEC, K_DEC)
    ed2 = jnp.pad(edges[:, 1], (0, PPAD - P)).reshape(PPAD // K_DEC, K_DEC)
    out = _decode(z, es2, ed2)
    return out[:P]


# speedup vs baseline: 3.5466x; 1.3949x over previous
import functools

import jax
import jax.numpy as jnp
from jax import lax
from jax.experimental import pallas as pl
from jax.experimental.pallas import tpu as pltpu
from jax.experimental.pallas import tpu_sc as plsc

N = 10000
E = 320000
P = 100000
IN_CH = 128
HIDDEN = 256

NC = 2
NS = 16
NPAD = 10240
RPT = NPAD // NS
F = 64

K_SEG = 80
CPT = 252
EPT = K_SEG * CPT
EPAD = EPT * NS
SJ = 36
SB = CPT // SJ

K_DEC = 48
DCH = 68
PPT = K_DEC * DCH
PPAD = PPT * NC * NS


def _fill(ref, val, rows, cols):
    v = jnp.full((16,), val, jnp.float32)

    def row(r, carry):
        def col(k, carry2):
            ref[r, pl.ds(k * 16, 16)] = v
            return carry2
        return lax.fori_loop(0, cols // 16, col, carry)

    lax.fori_loop(0, rows, row, 0)


def _fill_1d(ref, val, n):
    v = jnp.full((16,), val, jnp.float32)

    def it(k, carry):
        ref[pl.ds(k * 16, 16)] = v
        return carry

    lax.fori_loop(0, n // 16, it, 0)


def _make_segsum(nq, with_cnt):
    qpc = nq // NC
    mesh = plsc.VectorSubcoreMesh(core_axis_name="c", subcore_axis_name="s")

    out_type = [jax.ShapeDtypeStruct((nq, NPAD, F), jnp.float32)]
    if with_cnt:
        out_type.append(jax.ShapeDtypeStruct((NPAD,), jnp.float32))

    scratch = [
        pltpu.VMEM_SHARED((NPAD, F), jnp.float32),
        pltpu.VMEM_SHARED((NPAD, F), jnp.float32),
        pltpu.VMEM((SJ, K_SEG), jnp.int32),
        pltpu.VMEM((SJ, K_SEG), jnp.int32),
    ] + [pltpu.VMEM((K_SEG, F), jnp.float32) for _ in range(4)] \
      + [pltpu.SemaphoreType.DMA for _ in range(8)]
    if with_cnt:
        scratch += [
            pltpu.VMEM_SHARED((NPAD,), jnp.float32),
            pltpu.VMEM((K_SEG,), jnp.float32),
        ] + [pltpu.SemaphoreType.DMA for _ in range(4)]

    def body(tab_hbm, src_hbm, dst_hbm, out_hbm, *rest):
        if with_cnt:
            (cnt_hbm, tab_s, acc_s, sidx_blk, didx_blk,
             rv0, rv1, rv2, rv3,
             sg0, sg1, sg2, sg3, ss0, ss1, ss2, ss3,
             cnt_s, ones_v, sc0, sc1, sc2, sc3) = rest
        else:
            (tab_s, acc_s, sidx_blk, didx_blk,
             rv0, rv1, rv2, rv3,
             sg0, sg1, sg2, sg3, ss0, ss1, ss2, ss3) = rest
        c = lax.axis_index("c")
        s = lax.axis_index("s")
        r0 = s * RPT
        rvs = (rv0, rv1, rv2, rv3)
        sgs = (sg0, sg1, sg2, sg3)
        sss = (ss0, ss1, ss2, ss3)

        for qi in range(qpc):
            q = c * qpc + qi
            _fill(rv0, 0.0, K_SEG, F)
            pltpu.sync_copy(tab_hbm.at[q, pl.ds(r0, RPT)],
                            tab_s.at[pl.ds(r0, RPT)])
            for zb in range(RPT // K_SEG):
                pltpu.sync_copy(rv0,
                                acc_s.at[pl.ds(r0 + zb * K_SEG, K_SEG)])
            if with_cnt and qi == 0:
                _fill_1d(ones_v, 0.0, K_SEG)

                @pl.when(c == 0)
                def _():
                    for zb in range(RPT // K_SEG):
                        pltpu.sync_copy(
                            ones_v, cnt_s.at[pl.ds(r0 + zb * K_SEG, K_SEG)])
                _fill_1d(ones_v, 1.0, K_SEG)
            plsc.subcore_barrier()

            def sblk(u, carry):
                row0 = s * CPT + u * SJ
                pltpu.sync_copy(src_hbm.at[pl.ds(row0, SJ)], sidx_blk)
                pltpu.sync_copy(dst_hbm.at[pl.ds(row0, SJ)], didx_blk)

                def quad(i, carry2):
                    gs = []
                    for b in range(4):
                        gs.append(pltpu.async_copy(
                            tab_s.at[sidx_blk.at[4 * i + b]], rvs[b],
                            sgs[b]))
                    scs = []
                    for b in range(4):
                        gs[b].wait()
                        scs.append(pltpu.async_copy(
                            rvs[b], acc_s.at[didx_blk.at[4 * i + b]],
                            sss[b], add=True))
                    if with_cnt and qi == 0:
                        @pl.when(c == 0)
                        def _():
                            ccs = [pltpu.async_copy(
                                ones_v, cnt_s.at[didx_blk.at[4 * i + b]],
                                (sc0, sc1, sc2, sc3)[b], add=True)
                                for b in range(4)]
                            for cc in ccs:
                                cc.wait()
                    for sp in scs:
                        sp.wait()
                    return carry2

                lax.fori_loop(0, SJ // 4, quad, 0)
                return carry

            lax.fori_loop(0, SB, sblk, 0)
            plsc.subcore_barrier()

            pltpu.sync_copy(acc_s.at[pl.ds(r0, RPT)],
                            out_hbm.at[q, pl.ds(r0, RPT)])
            if with_cnt and qi == 0:
                @pl.when(c == 0)
                def _():
                    pltpu.sync_copy(cnt_s.at[pl.ds(r0, RPT)],
                                    cnt_hbm.at[pl.ds(r0, RPT)])

    return pl.kernel(body, out_type=tuple(out_type), mesh=mesh,
                     scratch_types=scratch,
                     compiler_params=pltpu.CompilerParams(
                         use_tc_tiling_on_sc=False))


_segsum2 = _make_segsum(2, True)
_segsum4 = _make_segsum(4, False)


RB = 512
NB = NPAD // RB


def _combine1_body(agg_ref, cnt_ref, x_ref, wl_ref, b_ref, wr_ref, out_ref):
    cnt = jnp.maximum(cnt_ref[...], 1.0)
    mean = jnp.concatenate([agg_ref[0], agg_ref[1]], axis=-1) / cnt
    h = (jnp.dot(mean, wl_ref[...], preferred_element_type=jnp.float32)
         + b_ref[...]
         + jnp.dot(x_ref[...], wr_ref[...],
                   preferred_element_type=jnp.float32))
    h = jnp.maximum(h, 0.0)
    for q in range(4):
        out_ref[q] = h[:, q * F:(q + 1) * F]


def _combine1(agg1, cnt2d, x_pad, W1_l, b1, W1_r):
    return pl.pallas_call(
        _combine1_body,
        grid=(NB,),
        in_specs=[
            pl.BlockSpec((2, RB, F), lambda i: (0, i, 0)),
            pl.BlockSpec((RB, 1), lambda i: (i, 0)),
            pl.BlockSpec((RB, IN_CH), lambda i: (i, 0)),
            pl.BlockSpec((IN_CH, HIDDEN), lambda i: (0, 0)),
            pl.BlockSpec((1, HIDDEN), lambda i: (0, 0)),
            pl.BlockSpec((IN_CH, HIDDEN), lambda i: (0, 0)),
        ],
        out_specs=pl.BlockSpec((4, RB, F), lambda i: (0, i, 0)),
        out_shape=jax.ShapeDtypeStruct((4, NPAD, F), jnp.float32),
    )(agg1, cnt2d, x_pad, W1_l, b1.reshape(1, HIDDEN), W1_r)


def _combine2_body(agg_ref, cnt_ref, h_ref, wl_ref, b_ref, wr_ref, out_ref):
    cnt = jnp.maximum(cnt_ref[...], 1.0)
    mean = jnp.concatenate([agg_ref[q] for q in range(4)], axis=-1) / cnt
    h = jnp.concatenate([h_ref[q] for q in range(4)], axis=-1)
    out_ref[...] = (
        jnp.dot(mean, wl_ref[...], preferred_element_type=jnp.float32)
        + b_ref[...]
        + jnp.dot(h, wr_ref[...], preferred_element_type=jnp.float32))


def _combine2(agg2, cnt2d, hT2, W2_l, b2, W2_r):
    return pl.pallas_call(
        _combine2_body,
        grid=(NB,),
        in_specs=[
            pl.BlockSpec((4, RB, F), lambda i: (0, i, 0)),
            pl.BlockSpec((RB, 1), lambda i: (i, 0)),
            pl.BlockSpec((4, RB, F), lambda i: (0, i, 0)),
            pl.BlockSpec((HIDDEN, HIDDEN), lambda i: (0, 0)),
            pl.BlockSpec((1, HIDDEN), lambda i: (0, 0)),
            pl.BlockSpec((HIDDEN, HIDDEN), lambda i: (0, 0)),
        ],
        out_specs=pl.BlockSpec((RB, HIDDEN), lambda i: (i, 0)),
        out_shape=jax.ShapeDtypeStruct((NPAD, HIDDEN), jnp.float32),
    )(agg2, cnt2d, hT2, W2_l, b2.reshape(1, HIDDEN), W2_r)


def _make_decode():
    mesh = plsc.VectorSubcoreMesh(core_axis_name="c", subcore_axis_name="s")
    scratch = (
        [pltpu.VMEM((DCH, K_DEC), jnp.int32) for _ in range(2)]
        + [pltpu.VMEM((K_DEC, HIDDEN), jnp.float32) for _ in range(8)]
        + [pltpu.VMEM((PPT,), jnp.float32)]
        + [pltpu.SemaphoreType.DMA for _ in range(8)]
    )

    def body(z_hbm, es_hbm, ed_hbm, out_hbm, sidx_all, didx_all,
             zs0, zd0, zs1, zd1, zs2, zd2, zs3, zd3, outv,
             ga0, gb0, ga1, gb1, ga2, gb2, ga3, gb3):
        c = lax.axis_index("c")
        s = lax.axis_index("s")
        w = c * NS + s
        lanes = lax.iota(jnp.int32, 16)
        z16 = jnp.zeros((16,), jnp.float32)
        zss = (zs0, zs1, zs2, zs3)
        zds = (zd0, zd1, zd2, zd3)
        gas = (ga0, ga1, ga2, ga3)
        gbs = (gb0, gb1, gb2, gb3)

        pltpu.sync_copy(es_hbm.at[pl.ds(w * DCH, DCH)], sidx_all)
        pltpu.sync_copy(ed_hbm.at[pl.ds(w * DCH, DCH)], didx_all)

        def compute(zs, zd, j):
            for g in range(K_DEC // 16):
                rows16 = lanes + (g * 16)

                def ki_loop(ki, accs):
                    a0, a1, a2, a3 = accs
                    kb = jnp.zeros((16,), jnp.int32) + ki * 16
                    for u in range(16):
                        kk = kb + u
                        va = plsc.load_gather(zs, [rows16, kk])
                        vb = plsc.load_gather(zd, [rows16, kk])
                        if u % 4 == 0:
                            a0 = a0 + va * vb
                        elif u % 4 == 1:
                            a1 = a1 + va * vb
                        elif u % 4 == 2:
                            a2 = a2 + va * vb
                        else:
                            a3 = a3 + va * vb
                    return (a0, a1, a2, a3)

                a0, a1, a2, a3 = lax.fori_loop(0, HIDDEN // 16, ki_loop,
                                               (z16, z16, z16, z16))
                outv[pl.ds(j * K_DEC + g * 16, 16)] = (a0 + a1) + (a2 + a3)

        def it(i, carry):
            cps = []
            for b in range(4):
                j = 4 * i + b
                cps.append((
                    pltpu.async_copy(z_hbm.at[sidx_all.at[j]], zss[b],
                                     gas[b]),
                    pltpu.async_copy(z_hbm.at[didx_all.at[j]], zds[b],
                                     gbs[b]),
                ))
            for b in range(4):
                cps[b][0].wait()
                cps[b][1].wait()
                compute(zss[b], zds[b], 4 * i + b)
            return carry

        lax.fori_loop(0, DCH // 4, it, 0)

        pltpu.sync_copy(outv, out_hbm.at[pl.ds(w * PPT, PPT)])

    return pl.kernel(body,
                     out_type=jax.ShapeDtypeStruct((PPAD,), jnp.float32),
                     mesh=mesh, scratch_types=scratch,
                     compiler_params=pltpu.CompilerParams(
                         use_tc_tiling_on_sc=False,
                         needs_layout_passes=False))


_decode = _make_decode()


def kernel(x, edge_index, edges, W1_l, b1, W1_r, W2_l, b2, W2_r):
    pad_e = EPAD - E
    pad_src = (jnp.arange(pad_e, dtype=jnp.int32) * 97) % N
    pad_dst = N + (jnp.arange(pad_e, dtype=jnp.int32) % (NPAD - N))
    src2 = jnp.concatenate([edge_index[0], pad_src]).reshape(
        EPAD // K_SEG, K_SEG)
    dst2 = jnp.concatenate([edge_index[1], pad_dst]).reshape(
        EPAD // K_SEG, K_SEG)
    x_pad = jnp.pad(x, ((0, NPAD - N), (0, 0)))
    xT2 = x_pad.reshape(NPAD, 2, F).transpose(1, 0, 2)

    agg1, cnt = _segsum2(xT2, src2, dst2)
    cnt2d = cnt.reshape(NPAD, 1)
    hT2 = _combine1(agg1, cnt2d, x_pad, W1_l, b1, W1_r)
    (agg2,) = _segsum4(hT2, src2, dst2)
    z = _combine2(agg2, cnt2d, hT2, W2_l, b2, W2_r)

    pad_p = PPAD - P
    pad_idx = (jnp.arange(pad_p, dtype=jnp.int32) * 89) % N
    es2 = jnp.concatenate([edges[:, 0], pad_idx]).reshape(
        PPAD // K_DEC, K_DEC)
    ed2 = jnp.concatenate([edges[:, 1], pad_idx]).reshape(
        PPAD // K_DEC, K_DEC)
    out = _decode(z, es2, ed2)
    return out[:P]


# speedup vs baseline: 5.8424x; 1.6473x over previous
import functools

import jax
import jax.numpy as jnp
from jax import lax
from jax.experimental import pallas as pl
from jax.experimental.pallas import tpu as pltpu
from jax.experimental.pallas import tpu_sc as plsc

N = 10000
E = 320000
P = 100000
IN_CH = 128
HIDDEN = 256

NC = 2
NS = 16
NPAD = 10240
RPT = NPAD // NS
F = 64

K_SEG = 80
CPT = 252
EPT = K_SEG * CPT
EPAD = EPT * NS
SJ = 36
SB = CPT // SJ

K_DEC = 48
DCH = 68
PPT = K_DEC * DCH
PPAD = PPT * NC * NS


def _fill(ref, val, rows, cols):
    v = jnp.full((16,), val, jnp.float32)

    def row(r, carry):
        def col(k, carry2):
            ref[r, pl.ds(k * 16, 16)] = v
            return carry2
        return lax.fori_loop(0, cols // 16, col, carry)

    lax.fori_loop(0, rows, row, 0)


def _fill_1d(ref, val, n):
    v = jnp.full((16,), val, jnp.float32)

    def it(k, carry):
        ref[pl.ds(k * 16, 16)] = v
        return carry

    lax.fori_loop(0, n // 16, it, 0)


def _make_segsum(nq, with_cnt):
    qpc = nq // NC
    mesh = plsc.VectorSubcoreMesh(core_axis_name="c", subcore_axis_name="s")

    out_type = [jax.ShapeDtypeStruct((nq, NPAD, F), jnp.float32)]
    if with_cnt:
        out_type.append(jax.ShapeDtypeStruct((NPAD,), jnp.float32))

    scratch = [
        pltpu.VMEM_SHARED((NPAD, F), jnp.float32),
        pltpu.VMEM_SHARED((NPAD, F), jnp.float32),
        pltpu.VMEM((SJ, K_SEG), jnp.int32),
        pltpu.VMEM((SJ, K_SEG), jnp.int32),
    ] + [pltpu.VMEM((K_SEG, F), jnp.float32) for _ in range(4)] \
      + [pltpu.SemaphoreType.DMA for _ in range(8)]
    if with_cnt:
        scratch += [
            pltpu.VMEM_SHARED((NPAD,), jnp.float32),
            pltpu.VMEM((K_SEG,), jnp.float32),
        ] + [pltpu.SemaphoreType.DMA for _ in range(4)]

    def body(tab_hbm, src_hbm, dst_hbm, out_hbm, *rest):
        if with_cnt:
            (cnt_hbm, tab_s, acc_s, sidx_blk, didx_blk,
             rv0, rv1, rv2, rv3,
             sg0, sg1, sg2, sg3, ss0, ss1, ss2, ss3,
             cnt_s, ones_v, sc0, sc1, sc2, sc3) = rest
        else:
            (tab_s, acc_s, sidx_blk, didx_blk,
             rv0, rv1, rv2, rv3,
             sg0, sg1, sg2, sg3, ss0, ss1, ss2, ss3) = rest
        c = lax.axis_index("c")
        s = lax.axis_index("s")
        r0 = s * RPT
        rvs = (rv0, rv1, rv2, rv3)
        sgs = (sg0, sg1, sg2, sg3)
        sss = (ss0, ss1, ss2, ss3)

        for qi in range(qpc):
            q = c * qpc + qi
            _fill(rv0, 0.0, K_SEG, F)
            pltpu.sync_copy(tab_hbm.at[q, pl.ds(r0, RPT)],
                            tab_s.at[pl.ds(r0, RPT)])
            for zb in range(RPT // K_SEG):
                pltpu.sync_copy(rv0,
                                acc_s.at[pl.ds(r0 + zb * K_SEG, K_SEG)])
            if with_cnt and qi == 0:
                _fill_1d(ones_v, 0.0, K_SEG)

                @pl.when(c == 0)
                def _():
                    for zb in range(RPT // K_SEG):
                        pltpu.sync_copy(
                            ones_v, cnt_s.at[pl.ds(r0 + zb * K_SEG, K_SEG)])
                _fill_1d(ones_v, 1.0, K_SEG)
            plsc.subcore_barrier()

            def sblk(u, carry):
                row0 = s * CPT + u * SJ
                pltpu.sync_copy(src_hbm.at[pl.ds(row0, SJ)], sidx_blk)
                pltpu.sync_copy(dst_hbm.at[pl.ds(row0, SJ)], didx_blk)

                def quad(i, carry2):
                    gs = []
                    for b in range(4):
                        gs.append(pltpu.async_copy(
                            tab_s.at[sidx_blk.at[4 * i + b]], rvs[b],
                            sgs[b]))
                    scs = []
                    for b in range(4):
                        gs[b].wait()
                        scs.append(pltpu.async_copy(
                            rvs[b], acc_s.at[didx_blk.at[4 * i + b]],
                            sss[b], add=True))
                    if with_cnt and qi == 0:
                        @pl.when(c == 0)
                        def _():
                            ccs = [pltpu.async_copy(
                                ones_v, cnt_s.at[didx_blk.at[4 * i + b]],
                                (sc0, sc1, sc2, sc3)[b], add=True)
                                for b in range(4)]
                            for cc in ccs:
                                cc.wait()
                    for sp in scs:
                        sp.wait()
                    return carry2

                lax.fori_loop(0, SJ // 4, quad, 0)
                return carry

            lax.fori_loop(0, SB, sblk, 0)
            plsc.subcore_barrier()

            pltpu.sync_copy(acc_s.at[pl.ds(r0, RPT)],
                            out_hbm.at[q, pl.ds(r0, RPT)])
            if with_cnt and qi == 0:
                @pl.when(c == 0)
                def _():
                    pltpu.sync_copy(cnt_s.at[pl.ds(r0, RPT)],
                                    cnt_hbm.at[pl.ds(r0, RPT)])

    return pl.kernel(body, out_type=tuple(out_type), mesh=mesh,
                     scratch_types=scratch,
                     compiler_params=pltpu.CompilerParams(
                         use_tc_tiling_on_sc=False))


_segsum2 = _make_segsum(2, True)
_segsum4 = _make_segsum(4, False)


RB = 512
NB = NPAD // RB


def _combine1_body(agg_ref, cnt_ref, x_ref, wl_ref, b_ref, wr_ref, out_ref):
    cnt = jnp.maximum(cnt_ref[...], 1.0)
    mean = jnp.concatenate([agg_ref[0], agg_ref[1]], axis=-1) / cnt
    h = (jnp.dot(mean, wl_ref[...], preferred_element_type=jnp.float32)
         + b_ref[...]
         + jnp.dot(x_ref[...], wr_ref[...],
                   preferred_element_type=jnp.float32))
    h = jnp.maximum(h, 0.0)
    for q in range(4):
        out_ref[q] = h[:, q * F:(q + 1) * F]


def _combine1(agg1, cnt2d, x_pad, W1_l, b1, W1_r):
    return pl.pallas_call(
        _combine1_body,
        grid=(NB,),
        in_specs=[
            pl.BlockSpec((2, RB, F), lambda i: (0, i, 0)),
            pl.BlockSpec((RB, 1), lambda i: (i, 0)),
            pl.BlockSpec((RB, IN_CH), lambda i: (i, 0)),
            pl.BlockSpec((IN_CH, HIDDEN), lambda i: (0, 0)),
            pl.BlockSpec((1, HIDDEN), lambda i: (0, 0)),
            pl.BlockSpec((IN_CH, HIDDEN), lambda i: (0, 0)),
        ],
        out_specs=pl.BlockSpec((4, RB, F), lambda i: (0, i, 0)),
        out_shape=jax.ShapeDtypeStruct((4, NPAD, F), jnp.float32),
    )(agg1, cnt2d, x_pad, W1_l, b1.reshape(1, HIDDEN), W1_r)


def _combine2_body(agg_ref, cnt_ref, h_ref, wl_ref, b_ref, wr_ref, out_ref):
    cnt = jnp.maximum(cnt_ref[...], 1.0)
    mean = jnp.concatenate([agg_ref[q] for q in range(4)], axis=-1) / cnt
    h = jnp.concatenate([h_ref[q] for q in range(4)], axis=-1)
    out_ref[...] = (
        jnp.dot(mean, wl_ref[...], preferred_element_type=jnp.float32)
        + b_ref[...]
        + jnp.dot(h, wr_ref[...], preferred_element_type=jnp.float32))


def _combine2(agg2, cnt2d, hT2, W2_l, b2, W2_r):
    return pl.pallas_call(
        _combine2_body,
        grid=(NB,),
        in_specs=[
            pl.BlockSpec((4, RB, F), lambda i: (0, i, 0)),
            pl.BlockSpec((RB, 1), lambda i: (i, 0)),
            pl.BlockSpec((4, RB, F), lambda i: (0, i, 0)),
            pl.BlockSpec((HIDDEN, HIDDEN), lambda i: (0, 0)),
            pl.BlockSpec((1, HIDDEN), lambda i: (0, 0)),
            pl.BlockSpec((HIDDEN, HIDDEN), lambda i: (0, 0)),
        ],
        out_specs=pl.BlockSpec((RB, HIDDEN), lambda i: (i, 0)),
        out_shape=jax.ShapeDtypeStruct((NPAD, HIDDEN), jnp.float32),
    )(agg2, cnt2d, hT2, W2_l, b2.reshape(1, HIDDEN), W2_r)


def _make_decode():
    mesh = plsc.VectorSubcoreMesh(core_axis_name="c", subcore_axis_name="s")
    scratch = (
        [pltpu.VMEM((DCH, K_DEC), jnp.int32) for _ in range(2)]
        + [pltpu.VMEM((K_DEC, HIDDEN), jnp.float32) for _ in range(8)]
        + [pltpu.VMEM((PPT,), jnp.float32)]
        + [pltpu.SemaphoreType.DMA for _ in range(8)]
    )

    def body(z_hbm, es_hbm, ed_hbm, out_hbm, sidx_all, didx_all,
             zs0, zd0, zs1, zd1, zs2, zd2, zs3, zd3, outv,
             ga0, gb0, ga1, gb1, ga2, gb2, ga3, gb3):
        c = lax.axis_index("c")
        s = lax.axis_index("s")
        w = c * NS + s
        lanes = lax.iota(jnp.int32, 16)
        z16 = jnp.zeros((16,), jnp.float32)
        zss = (zs0, zs1, zs2, zs3)
        zds = (zd0, zd1, zd2, zd3)
        gas = (ga0, ga1, ga2, ga3)
        gbs = (gb0, gb1, gb2, gb3)

        pltpu.sync_copy(es_hbm.at[pl.ds(w * DCH, DCH)], sidx_all)
        pltpu.sync_copy(ed_hbm.at[pl.ds(w * DCH, DCH)], didx_all)

        def compute(zs, zd, j):
            def pair_loop(g, carry):
                vec = z16
                for i in range(16):
                    idx = g * 16 + i
                    terms = []
                    for t in range(HIDDEN // 16):
                        va = zs[idx, pl.ds(t * 16, 16)]
                        vb = zd[idx, pl.ds(t * 16, 16)]
                        terms.append(va * vb)
                    while len(terms) > 1:
                        terms = [terms[k] + terms[k + 1]
                                 for k in range(0, len(terms) - 1, 2)] + (
                                     [terms[-1]] if len(terms) % 2 else [])
                    vec = jnp.where(lanes == i, jnp.sum(terms[0]), vec)
                outv[pl.ds(j * K_DEC + g * 16, 16)] = vec
                return carry

            lax.fori_loop(0, K_DEC // 16, pair_loop, 0)

        def it(i, carry):
            cps = []
            for b in range(4):
                j = 4 * i + b
                cps.append((
                    pltpu.async_copy(z_hbm.at[sidx_all.at[j]], zss[b],
                                     gas[b]),
                    pltpu.async_copy(z_hbm.at[didx_all.at[j]], zds[b],
                                     gbs[b]),
                ))
            for b in range(4):
                cps[b][0].wait()
                cps[b][1].wait()
                compute(zss[b], zds[b], 4 * i + b)
            return carry

        lax.fori_loop(0, DCH // 4, it, 0)

        pltpu.sync_copy(outv, out_hbm.at[pl.ds(w * PPT, PPT)])

    return pl.kernel(body,
                     out_type=jax.ShapeDtypeStruct((PPAD,), jnp.float32),
                     mesh=mesh, scratch_types=scratch,
                     compiler_params=pltpu.CompilerParams(
                         use_tc_tiling_on_sc=False,
                         needs_layout_passes=False))


_decode = _make_decode()


def kernel(x, edge_index, edges, W1_l, b1, W1_r, W2_l, b2, W2_r):
    pad_e = EPAD - E
    pad_src = (jnp.arange(pad_e, dtype=jnp.int32) * 97) % N
    pad_dst = N + (jnp.arange(pad_e, dtype=jnp.int32) % (NPAD - N))
    src2 = jnp.concatenate([edge_index[0], pad_src]).reshape(
        EPAD // K_SEG, K_SEG)
    dst2 = jnp.concatenate([edge_index[1], pad_dst]).reshape(
        EPAD // K_SEG, K_SEG)
    x_pad = jnp.pad(x, ((0, NPAD - N), (0, 0)))
    xT2 = x_pad.reshape(NPAD, 2, F).transpose(1, 0, 2)

    agg1, cnt = _segsum2(xT2, src2, dst2)
    cnt2d = cnt.reshape(NPAD, 1)
    hT2 = _combine1(agg1, cnt2d, x_pad, W1_l, b1, W1_r)
    (agg2,) = _segsum4(hT2, src2, dst2)
    z = _combine2(agg2, cnt2d, hT2, W2_l, b2, W2_r)

    pad_p = PPAD - P
    pad_idx = (jnp.arange(pad_p, dtype=jnp.int32) * 89) % N
    es2 = jnp.concatenate([edges[:, 0], pad_idx]).reshape(
        PPAD // K_DEC, K_DEC)
    ed2 = jnp.concatenate([edges[:, 1], pad_idx]).reshape(
        PPAD // K_DEC, K_DEC)
    out = _decode(z, es2, ed2)
    return out[:P]


# speedup vs baseline: 7.2690x; 1.2442x over previous
import functools

import jax
import jax.numpy as jnp
from jax import lax
from jax.experimental import pallas as pl
from jax.experimental.pallas import tpu as pltpu
from jax.experimental.pallas import tpu_sc as plsc

N = 10000
E = 320000
P = 100000
IN_CH = 128
HIDDEN = 256

NC = 2
NS = 16
NPAD = 10240
RPT = NPAD // NS
F = 64

K_SEG = 80
CPT = 252
EPT = K_SEG * CPT
EPAD = EPT * NS
SJ = 36
SB = CPT // SJ

K_DEC = 96
DCH = 34
PPT = K_DEC * DCH
PPAD = PPT * NC * NS


def _fill(ref, val, rows, cols):
    v = jnp.full((16,), val, jnp.float32)

    def row(r, carry):
        def col(k, carry2):
            ref[r, pl.ds(k * 16, 16)] = v
            return carry2
        return lax.fori_loop(0, cols // 16, col, carry)

    lax.fori_loop(0, rows, row, 0)


def _fill_1d(ref, val, n):
    v = jnp.full((16,), val, jnp.float32)

    def it(k, carry):
        ref[pl.ds(k * 16, 16)] = v
        return carry

    lax.fori_loop(0, n // 16, it, 0)


def _make_segsum(nq, with_cnt):
    qpc = nq // NC
    mesh = plsc.VectorSubcoreMesh(core_axis_name="c", subcore_axis_name="s")

    out_type = [jax.ShapeDtypeStruct((nq, NPAD, F), jnp.float32)]
    if with_cnt:
        out_type.append(jax.ShapeDtypeStruct((NPAD,), jnp.float32))

    scratch = [
        pltpu.VMEM_SHARED((NPAD, F), jnp.float32),
        pltpu.VMEM_SHARED((NPAD, F), jnp.float32),
        pltpu.VMEM((SJ, K_SEG), jnp.int32),
        pltpu.VMEM((SJ, K_SEG), jnp.int32),
    ] + [pltpu.VMEM((K_SEG, F), jnp.float32) for _ in range(4)] \
      + [pltpu.SemaphoreType.DMA for _ in range(8)]
    if with_cnt:
        scratch += [
            pltpu.VMEM_SHARED((NPAD,), jnp.float32),
            pltpu.VMEM((K_SEG,), jnp.float32),
        ] + [pltpu.SemaphoreType.DMA for _ in range(4)]

    def body(tab_hbm, src_hbm, dst_hbm, out_hbm, *rest):
        if with_cnt:
            (cnt_hbm, tab_s, acc_s, sidx_blk, didx_blk,
             rv0, rv1, rv2, rv3,
             sg0, sg1, sg2, sg3, ss0, ss1, ss2, ss3,
             cnt_s, ones_v, sc0, sc1, sc2, sc3) = rest
        else:
            (tab_s, acc_s, sidx_blk, didx_blk,
             rv0, rv1, rv2, rv3,
             sg0, sg1, sg2, sg3, ss0, ss1, ss2, ss3) = rest
        c = lax.axis_index("c")
        s = lax.axis_index("s")
        r0 = s * RPT
        rvs = (rv0, rv1, rv2, rv3)
        sgs = (sg0, sg1, sg2, sg3)
        sss = (ss0, ss1, ss2, ss3)

        for qi in range(qpc):
            q = c * qpc + qi
            _fill(rv0, 0.0, K_SEG, F)
            pltpu.sync_copy(tab_hbm.at[q, pl.ds(r0, RPT)],
                            tab_s.at[pl.ds(r0, RPT)])
            for zb in range(RPT // K_SEG):
                pltpu.sync_copy(rv0,
                                acc_s.at[pl.ds(r0 + zb * K_SEG, K_SEG)])
            if with_cnt and qi == 0:
                _fill_1d(ones_v, 0.0, K_SEG)

                @pl.when(c == 0)
                def _():
                    for zb in range(RPT // K_SEG):
                        pltpu.sync_copy(
                            ones_v, cnt_s.at[pl.ds(r0 + zb * K_SEG, K_SEG)])
                _fill_1d(ones_v, 1.0, K_SEG)
            plsc.subcore_barrier()

            def sblk(u, carry):
                row0 = s * CPT + u * SJ
                pltpu.sync_copy(src_hbm.at[pl.ds(row0, SJ)], sidx_blk)
                pltpu.sync_copy(dst_hbm.at[pl.ds(row0, SJ)], didx_blk)

                def quad(i, carry2):
                    gs = []
                    for b in range(4):
                        gs.append(pltpu.async_copy(
                            tab_s.at[sidx_blk.at[4 * i + b]], rvs[b],
                            sgs[b]))
                    scs = []
                    for b in range(4):
                        gs[b].wait()
                        scs.append(pltpu.async_copy(
                            rvs[b], acc_s.at[didx_blk.at[4 * i + b]],
                            sss[b], add=True))
                    if with_cnt and qi == 0:
                        @pl.when(c == 0)
                        def _():
                            ccs = [pltpu.async_copy(
                                ones_v, cnt_s.at[didx_blk.at[4 * i + b]],
                                (sc0, sc1, sc2, sc3)[b], add=True)
                                for b in range(4)]
                            for cc in ccs:
                                cc.wait()
                    for sp in scs:
                        sp.wait()
                    return carry2

                lax.fori_loop(0, SJ // 4, quad, 0)
                return carry

            lax.fori_loop(0, SB, sblk, 0)
            plsc.subcore_barrier()

            pltpu.sync_copy(acc_s.at[pl.ds(r0, RPT)],
                            out_hbm.at[q, pl.ds(r0, RPT)])
            if with_cnt and qi == 0:
                @pl.when(c == 0)
                def _():
                    pltpu.sync_copy(cnt_s.at[pl.ds(r0, RPT)],
                                    cnt_hbm.at[pl.ds(r0, RPT)])

    return pl.kernel(body, out_type=tuple(out_type), mesh=mesh,
                     scratch_types=scratch,
                     compiler_params=pltpu.CompilerParams(
                         use_tc_tiling_on_sc=False))


_segsum2 = _make_segsum(2, True)
_segsum4 = _make_segsum(4, False)


RB = 512
NB = NPAD // RB


def _combine1_body(agg_ref, cnt_ref, x_ref, wl_ref, b_ref, wr_ref, out_ref):
    cnt = jnp.maximum(cnt_ref[...], 1.0)
    mean = jnp.concatenate([agg_ref[0], agg_ref[1]], axis=-1) / cnt
    h = (jnp.dot(mean, wl_ref[...], preferred_element_type=jnp.float32)
         + b_ref[...]
         + jnp.dot(x_ref[...], wr_ref[...],
                   preferred_element_type=jnp.float32))
    h = jnp.maximum(h, 0.0)
    for q in range(4):
        out_ref[q] = h[:, q * F:(q + 1) * F]


def _combine1(agg1, cnt2d, x_pad, W1_l, b1, W1_r):
    return pl.pallas_call(
        _combine1_body,
        grid=(NB,),
        in_specs=[
            pl.BlockSpec((2, RB, F), lambda i: (0, i, 0)),
            pl.BlockSpec((RB, 1), lambda i: (i, 0)),
            pl.BlockSpec((RB, IN_CH), lambda i: (i, 0)),
            pl.BlockSpec((IN_CH, HIDDEN), lambda i: (0, 0)),
            pl.BlockSpec((1, HIDDEN), lambda i: (0, 0)),
            pl.BlockSpec((IN_CH, HIDDEN), lambda i: (0, 0)),
        ],
        out_specs=pl.BlockSpec((4, RB, F), lambda i: (0, i, 0)),
        out_shape=jax.ShapeDtypeStruct((4, NPAD, F), jnp.float32),
    )(agg1, cnt2d, x_pad, W1_l, b1.reshape(1, HIDDEN), W1_r)


def _combine2_body(agg_ref, cnt_ref, h_ref, wl_ref, b_ref, wr_ref, out_ref):
    cnt = jnp.maximum(cnt_ref[...], 1.0)
    mean = jnp.concatenate([agg_ref[q] for q in range(4)], axis=-1) / cnt
    h = jnp.concatenate([h_ref[q] for q in range(4)], axis=-1)
    out_ref[...] = (
        jnp.dot(mean, wl_ref[...], preferred_element_type=jnp.float32)
        + b_ref[...]
        + jnp.dot(h, wr_ref[...], preferred_element_type=jnp.float32)
    ).astype(jnp.bfloat16)


def _combine2(agg2, cnt2d, hT2, W2_l, b2, W2_r):
    return pl.pallas_call(
        _combine2_body,
        grid=(NB,),
        in_specs=[
            pl.BlockSpec((4, RB, F), lambda i: (0, i, 0)),
            pl.BlockSpec((RB, 1), lambda i: (i, 0)),
            pl.BlockSpec((4, RB, F), lambda i: (0, i, 0)),
            pl.BlockSpec((HIDDEN, HIDDEN), lambda i: (0, 0)),
            pl.BlockSpec((1, HIDDEN), lambda i: (0, 0)),
            pl.BlockSpec((HIDDEN, HIDDEN), lambda i: (0, 0)),
        ],
        out_specs=pl.BlockSpec((RB, HIDDEN), lambda i: (i, 0)),
        out_shape=jax.ShapeDtypeStruct((NPAD, HIDDEN), jnp.bfloat16),
    )(agg2, cnt2d, hT2, W2_l, b2.reshape(1, HIDDEN), W2_r)


def _make_decode():
    mesh = plsc.VectorSubcoreMesh(core_axis_name="c", subcore_axis_name="s")
    scratch = (
        [pltpu.VMEM((DCH, K_DEC), jnp.int32) for _ in range(2)]
        + [pltpu.VMEM((K_DEC, HIDDEN), jnp.bfloat16) for _ in range(8)]
        + [pltpu.VMEM((PPT,), jnp.float32)]
        + [pltpu.SemaphoreType.DMA for _ in range(8)]
    )

    def body(z_hbm, es_hbm, ed_hbm, out_hbm, sidx_all, didx_all,
             zs0, zd0, zs1, zd1, zs2, zd2, zs3, zd3, outv,
             ga0, gb0, ga1, gb1, ga2, gb2, ga3, gb3):
        c = lax.axis_index("c")
        s = lax.axis_index("s")
        w = c * NS + s
        lanes = lax.iota(jnp.int32, 16)
        z16 = jnp.zeros((16,), jnp.float32)
        zss = (zs0, zs1, zs2, zs3)
        zds = (zd0, zd1, zd2, zd3)
        gas = (ga0, ga1, ga2, ga3)
        gbs = (gb0, gb1, gb2, gb3)

        pltpu.sync_copy(es_hbm.at[pl.ds(w * DCH, DCH)], sidx_all)
        pltpu.sync_copy(ed_hbm.at[pl.ds(w * DCH, DCH)], didx_all)

        def compute(zs, zd, j):
            def pair_loop(g, carry):
                vec = z16
                for i in range(16):
                    idx = g * 16 + i
                    terms = []
                    for t in range(HIDDEN // 32):
                        va = zs[idx, pl.ds(t * 32, 32)]
                        vb = zd[idx, pl.ds(t * 32, 32)]
                        terms.append(va * vb)
                    while len(terms) > 1:
                        terms = [terms[k] + terms[k + 1]
                                 for k in range(0, len(terms) - 1, 2)] + (
                                     [terms[-1]] if len(terms) % 2 else [])
                    lo, hi = plsc.unpack(terms[0],
                                         format=plsc.PackFormat.INTERLEAVED)
                    vec = jnp.where(lanes == i, jnp.sum(lo + hi), vec)
                outv[pl.ds(j * K_DEC + g * 16, 16)] = vec
                return carry

            lax.fori_loop(0, K_DEC // 16, pair_loop, 0)

        def it(i, carry):
            cps = []
            for b in range(4):
                j = 4 * i + b
                cps.append((
                    pltpu.async_copy(z_hbm.at[sidx_all.at[j]], zss[b],
                                     gas[b]),
                    pltpu.async_copy(z_hbm.at[didx_all.at[j]], zds[b],
                                     gbs[b]),
                ))
            for b in range(4):
                cps[b][0].wait()
                cps[b][1].wait()
                compute(zss[b], zds[b], 4 * i + b)
            return carry

        lax.fori_loop(0, DCH // 4, it, 0)

        pltpu.sync_copy(outv, out_hbm.at[pl.ds(w * PPT, PPT)])

    return pl.kernel(body,
                     out_type=jax.ShapeDtypeStruct((PPAD,), jnp.float32),
                     mesh=mesh, scratch_types=scratch,
                     compiler_params=pltpu.CompilerParams(
                         use_tc_tiling_on_sc=False,
                         needs_layout_passes=False))


_decode = _make_decode()


def kernel(x, edge_index, edges, W1_l, b1, W1_r, W2_l, b2, W2_r):
    pad_e = EPAD - E
    pad_src = (jnp.arange(pad_e, dtype=jnp.int32) * 97) % N
    pad_dst = N + (jnp.arange(pad_e, dtype=jnp.int32) % (NPAD - N))
    src2 = jnp.concatenate([edge_index[0], pad_src]).reshape(
        EPAD // K_SEG, K_SEG)
    dst2 = jnp.concatenate([edge_index[1], pad_dst]).reshape(
        EPAD // K_SEG, K_SEG)
    x_pad = jnp.pad(x, ((0, NPAD - N), (0, 0)))
    xT2 = x_pad.reshape(NPAD, 2, F).transpose(1, 0, 2)

    agg1, cnt = _segsum2(xT2, src2, dst2)
    cnt2d = cnt.reshape(NPAD, 1)
    hT2 = _combine1(agg1, cnt2d, x_pad, W1_l, b1, W1_r)
    (agg2,) = _segsum4(hT2, src2, dst2)
    z = _combine2(agg2, cnt2d, hT2, W2_l, b2, W2_r)

    pad_p = PPAD - P
    pad_idx = (jnp.arange(pad_p, dtype=jnp.int32) * 89) % N
    es2 = jnp.concatenate([edges[:, 0], pad_idx]).reshape(
        PPAD // K_DEC, K_DEC)
    ed2 = jnp.concatenate([edges[:, 1], pad_idx]).reshape(
        PPAD // K_DEC, K_DEC)
    out = _decode(z, es2, ed2)
    return out[:P]
